# baseline (speedup 1.0000x reference)
import functools

import jax
import jax.numpy as jnp
from jax import lax
from jax.experimental import pallas as pl
from jax.experimental.pallas import tpu as pltpu
from jax.experimental.pallas import tpu_sc as plsc

S, D, F, E = 2048, 768, 256, 8
BLK = 128
NB = 24
NSLOT = NB * BLK
NW = 32


def _router(xt, Wr):
    def body(x_ref, wr_ref, out_ref):
        logits = jnp.dot(x_ref[...], wr_ref[...],
                         preferred_element_type=jnp.float32)
        out_ref[...] = jnp.argmax(logits, axis=1).astype(jnp.int32)

    return pl.pallas_call(
        body,
        out_shape=jax.ShapeDtypeStruct((S,), jnp.int32),
    )(xt, Wr)


def _sc_gather(table, idx, nrows):
    bpw = nrows // NW
    d = table.shape[1]
    mesh = plsc.VectorSubcoreMesh(core_axis_name="c", subcore_axis_name="s",
                                  num_cores=2, num_subcores=16)

    @functools.partial(
        pl.kernel,
        mesh=mesh,
        out_type=jax.ShapeDtypeStruct((nrows, d), jnp.float32),
        scratch_types=[
            pltpu.VMEM((bpw,), jnp.int32),
            pltpu.VMEM((bpw, d), jnp.float32),
            pltpu.SemaphoreType.DMA,
        ],
    )
    def k(table_hbm, idx_hbm, out_hbm, idx_v, rows_v, sem):
        wid = lax.axis_index("s") * 2 + lax.axis_index("c")
        base = wid * bpw
        pltpu.sync_copy(idx_hbm.at[pl.ds(base, bpw)], idx_v)
        pltpu.async_copy(table_hbm.at[idx_v], rows_v, sem).wait()
        pltpu.sync_copy(rows_v, out_hbm.at[pl.ds(base, bpw)])

    return k(table, idx)


def _grouped_mlp(block_expert, xs, Wg, Wu, Wd):
    def body(be_ref, xs_ref, wg_ref, wu_ref, wd_ref, out_ref):
        xb = xs_ref[...]
        g = jnp.dot(xb, wg_ref[0], preferred_element_type=jnp.float32)
        u = jnp.dot(xb, wu_ref[0], preferred_element_type=jnp.float32)
        h = g * lax.logistic(g) * u
        out_ref[...] = jnp.dot(h, wd_ref[0], preferred_element_type=jnp.float32)

    grid_spec = pltpu.PrefetchScalarGridSpec(
        num_scalar_prefetch=1,
        grid=(NB,),
        in_specs=[
            pl.BlockSpec((BLK, D), lambda i, be: (i, 0)),
            pl.BlockSpec((1, D, F), lambda i, be: (be[i], 0, 0)),
            pl.BlockSpec((1, D, F), lambda i, be: (be[i], 0, 0)),
            pl.BlockSpec((1, F, D), lambda i, be: (be[i], 0, 0)),
        ],
        out_specs=pl.BlockSpec((BLK, D), lambda i, be: (i, 0)),
    )
    return pl.pallas_call(
        body,
        grid_spec=grid_spec,
        out_shape=jax.ShapeDtypeStruct((NSLOT, D), jnp.float32),
    )(block_expert, xs, Wg, Wu, Wd)


def kernel(x, Wr, Wg, Wu, Wd):
    b, s, d = x.shape
    xt = x.reshape(s, d)

    e_star = _router(xt, Wr)

    onehot = (e_star[:, None] == jnp.arange(E, dtype=jnp.int32)[None, :])
    onehot = onehot.astype(jnp.int32)
    ranks = jnp.cumsum(onehot, axis=0) - 1
    rank = jnp.take_along_axis(ranks, e_star[:, None], axis=1)[:, 0]
    counts = ranks[-1] + 1
    padded = ((counts + BLK - 1) // BLK) * BLK
    starts = jnp.concatenate(
        [jnp.zeros((1,), jnp.int32), jnp.cumsum(padded)[:-1]])
    position = starts[e_star] + rank
    gather_idx = jnp.zeros((NSLOT,), jnp.int32).at[position].set(
        jnp.arange(S, dtype=jnp.int32))
    sblk = starts // BLK
    block_expert = jnp.sum(
        jnp.arange(NB, dtype=jnp.int32)[:, None] >= sblk[None, :],
        axis=1) - 1
    block_expert = jnp.clip(block_expert, 0, E - 1)

    xs = _sc_gather(xt, gather_idx, NSLOT)
    ys = _grouped_mlp(block_expert, xs, Wg, Wu, Wd)
    out = _sc_gather(ys, position, S)
    return out.reshape(b, s, d)


# speedup vs baseline: 1.5480x; 1.5480x over previous
import functools

import jax
import jax.numpy as jnp
from jax import lax
from jax.experimental import pallas as pl
from jax.experimental.pallas import tpu as pltpu
from jax.experimental.pallas import tpu_sc as plsc

S, D, F, E = 2048, 768, 256, 8
BLK = 128
NB = 24
NSLOT = NB * BLK
NW = 32


def _router(xt, Wr):
    def body(x_ref, wr_ref, out_ref):
        logits = jnp.dot(x_ref[...], wr_ref[...],
                         preferred_element_type=jnp.float32)
        out_ref[...] = jnp.argmax(logits, axis=1).astype(jnp.int32)

    return pl.pallas_call(
        body,
        out_shape=jax.ShapeDtypeStruct((S,), jnp.int32),
    )(xt, Wr)


def _sc_gather(table, idx, nrows):
    bpw = nrows // NW
    d = table.shape[1]
    mesh = plsc.VectorSubcoreMesh(core_axis_name="c", subcore_axis_name="s",
                                  num_cores=2, num_subcores=16)

    @functools.partial(
        pl.kernel,
        mesh=mesh,
        out_type=jax.ShapeDtypeStruct((nrows, d), jnp.float32),
        scratch_types=[
            pltpu.VMEM((bpw,), jnp.int32),
            pltpu.VMEM((bpw, d), jnp.float32),
            pltpu.SemaphoreType.DMA,
        ],
    )
    def k(table_hbm, idx_hbm, out_hbm, idx_v, rows_v, sem):
        wid = lax.axis_index("s") * 2 + lax.axis_index("c")
        base = wid * bpw
        pltpu.sync_copy(idx_hbm.at[pl.ds(base, bpw)], idx_v)
        pltpu.async_copy(table_hbm.at[idx_v], rows_v, sem).wait()
        pltpu.sync_copy(rows_v, out_hbm.at[pl.ds(base, bpw)])

    return k(table, idx)


def _grouped_mlp(block_expert, xs, Wg, Wu, Wd):
    def body(be_ref, xs_ref, wg_ref, wu_ref, wd_ref, out_ref):
        xb = xs_ref[...]
        g = jnp.dot(xb, wg_ref[0], preferred_element_type=jnp.float32)
        u = jnp.dot(xb, wu_ref[0], preferred_element_type=jnp.float32)
        h = g * lax.logistic(g) * u
        out_ref[...] = jnp.dot(h, wd_ref[0], preferred_element_type=jnp.float32)

    grid_spec = pltpu.PrefetchScalarGridSpec(
        num_scalar_prefetch=1,
        grid=(NB,),
        in_specs=[
            pl.BlockSpec((BLK, D), lambda i, be: (i, 0)),
            pl.BlockSpec((1, D, F), lambda i, be: (be[i], 0, 0)),
            pl.BlockSpec((1, D, F), lambda i, be: (be[i], 0, 0)),
            pl.BlockSpec((1, F, D), lambda i, be: (be[i], 0, 0)),
        ],
        out_specs=pl.BlockSpec((BLK, D), lambda i, be: (i, 0)),
    )
    return pl.pallas_call(
        body,
        grid_spec=grid_spec,
        out_shape=jax.ShapeDtypeStruct((NSLOT, D), jnp.float32),
    )(block_expert, xs, Wg, Wu, Wd)


def kernel(x, Wr, Wg, Wu, Wd):
    b, s, d = x.shape
    xt = x.reshape(s, d)

    e_star = _router(xt, Wr)

    onehot = (e_star[:, None] == jnp.arange(E, dtype=jnp.int32)[None, :])
    onehot = onehot.astype(jnp.int32)
    ranks = jnp.cumsum(onehot, axis=0) - 1
    rank = jnp.take_along_axis(ranks, e_star[:, None], axis=1)[:, 0]
    counts = ranks[-1] + 1
    padded = ((counts + BLK - 1) // BLK) * BLK
    starts = jnp.concatenate(
        [jnp.zeros((1,), jnp.int32), jnp.cumsum(padded)[:-1]])
    position = starts[e_star] + rank
    gather_idx = (jnp.arange(NSLOT, dtype=jnp.int32) % S).at[position].set(
        jnp.arange(S, dtype=jnp.int32))
    sblk = starts // BLK
    block_expert = jnp.sum(
        jnp.arange(NB, dtype=jnp.int32)[:, None] >= sblk[None, :],
        axis=1) - 1
    block_expert = jnp.clip(block_expert, 0, E - 1)

    xs = _sc_gather(xt, gather_idx, NSLOT)
    ys = _grouped_mlp(block_expert, xs, Wg, Wu, Wd)
    out = _sc_gather(ys, position, S)
    return out.reshape(b, s, d)


# speedup vs baseline: 2.1708x; 1.4024x over previous
import functools

import jax
import jax.numpy as jnp
from jax import lax
from jax.experimental import pallas as pl
from jax.experimental.pallas import tpu as pltpu
from jax.experimental.pallas import tpu_sc as plsc

S, D, F, E = 2048, 768, 256, 8
BLK = 128
NB = 24
NSLOT = NB * BLK
NW = 32


def _plan(xt, Wr):
    def body(x_ref, wr_ref, pos_ref, be_ref):
        logits = jnp.dot(x_ref[...], wr_ref[...],
                         preferred_element_type=jnp.float32)
        iota_e = lax.broadcasted_iota(jnp.int32, (S, E), 1).astype(jnp.float32)
        max_l = jnp.max(logits, axis=1, keepdims=True)
        eqm = (logits == max_l).astype(jnp.float32)
        e_star = (E - 1) - jnp.max(eqm * ((E - 1) - iota_e), axis=1,
                                   keepdims=True)
        oh = (iota_e == e_star).astype(jnp.float32)
        ii = lax.broadcasted_iota(jnp.int32, (BLK, BLK), 0)
        jj = lax.broadcasted_iota(jnp.int32, (BLK, BLK), 1)
        tril = (ii >= jj).astype(jnp.float32)
        parts = []
        acc = jnp.zeros((1, E), jnp.float32)
        for c in range(S // BLK):
            part = jnp.dot(tril, oh[c * BLK:(c + 1) * BLK, :],
                           preferred_element_type=jnp.float32)
            parts.append(part + acc)
            acc = acc + part[BLK - 1:BLK, :]
        ranks_incl = jnp.concatenate(parts, axis=0)
        counts = acc
        padded = jnp.ceil(counts * (1.0 / BLK)) * BLK
        ee = lax.broadcasted_iota(jnp.int32, (E, E), 0)
        ff = lax.broadcasted_iota(jnp.int32, (E, E), 1)
        lt_strict = (ee < ff).astype(jnp.float32)
        starts = jnp.dot(padded, lt_strict,
                         preferred_element_type=jnp.float32)
        pos = jnp.sum((starts + ranks_incl - 1.0) * oh, axis=1)
        pos_ref[...] = pos.astype(jnp.int32)
        sblk = starts * (1.0 / BLK)
        nb_iota = lax.broadcasted_iota(jnp.int32, (NB, E), 0).astype(jnp.float32)
        be = jnp.sum((nb_iota >= sblk).astype(jnp.float32), axis=1) - 1.0
        be_ref[...] = jnp.clip(be, 0.0, E - 1.0).astype(jnp.int32)

    return pl.pallas_call(
        body,
        out_shape=[
            jax.ShapeDtypeStruct((S,), jnp.int32),
            jax.ShapeDtypeStruct((NB,), jnp.int32),
        ],
    )(xt, Wr)


def _sc_scatter(xt, position):
    bpw = S // NW
    mesh = plsc.VectorSubcoreMesh(core_axis_name="c", subcore_axis_name="s",
                                  num_cores=2, num_subcores=16)

    @functools.partial(
        pl.kernel,
        mesh=mesh,
        out_type=jax.ShapeDtypeStruct((NSLOT, D), jnp.float32),
        scratch_types=[
            pltpu.VMEM((bpw,), jnp.int32),
            pltpu.VMEM((bpw, D), jnp.float32),
            pltpu.SemaphoreType.DMA,
        ],
    )
    def k(x_hbm, pos_hbm, out_hbm, idx_v, rows_v, sem):
        wid = lax.axis_index("s") * 2 + lax.axis_index("c")
        base = wid * bpw
        pltpu.sync_copy(pos_hbm.at[pl.ds(base, bpw)], idx_v)
        pltpu.sync_copy(x_hbm.at[pl.ds(base, bpw)], rows_v)
        pltpu.async_copy(rows_v, out_hbm.at[idx_v], sem).wait()

    return k(xt, position)


def _sc_gather(table, idx, nrows):
    bpw = nrows // NW
    d = table.shape[1]
    mesh = plsc.VectorSubcoreMesh(core_axis_name="c", subcore_axis_name="s",
                                  num_cores=2, num_subcores=16)

    @functools.partial(
        pl.kernel,
        mesh=mesh,
        out_type=jax.ShapeDtypeStruct((nrows, d), jnp.float32),
        scratch_types=[
            pltpu.VMEM((bpw,), jnp.int32),
            pltpu.VMEM((bpw, d), jnp.float32),
            pltpu.SemaphoreType.DMA,
        ],
    )
    def k(table_hbm, idx_hbm, out_hbm, idx_v, rows_v, sem):
        wid = lax.axis_index("s") * 2 + lax.axis_index("c")
        base = wid * bpw
        pltpu.sync_copy(idx_hbm.at[pl.ds(base, bpw)], idx_v)
        pltpu.async_copy(table_hbm.at[idx_v], rows_v, sem).wait()
        pltpu.sync_copy(rows_v, out_hbm.at[pl.ds(base, bpw)])

    return k(table, idx)


def _grouped_mlp(block_expert, xs, Wg, Wu, Wd):
    def body(be_ref, xs_ref, wg_ref, wu_ref, wd_ref, out_ref):
        xb = xs_ref[...]
        g = jnp.dot(xb, wg_ref[0], preferred_element_type=jnp.float32)
        u = jnp.dot(xb, wu_ref[0], preferred_element_type=jnp.float32)
        h = g * lax.logistic(g) * u
        out_ref[...] = jnp.dot(h, wd_ref[0], preferred_element_type=jnp.float32)

    grid_spec = pltpu.PrefetchScalarGridSpec(
        num_scalar_prefetch=1,
        grid=(NB,),
        in_specs=[
            pl.BlockSpec((BLK, D), lambda i, be: (i, 0)),
            pl.BlockSpec((1, D, F), lambda i, be: (be[i], 0, 0)),
            pl.BlockSpec((1, D, F), lambda i, be: (be[i], 0, 0)),
            pl.BlockSpec((1, F, D), lambda i, be: (be[i], 0, 0)),
        ],
        out_specs=pl.BlockSpec((BLK, D), lambda i, be: (i, 0)),
    )
    return pl.pallas_call(
        body,
        grid_spec=grid_spec,
        out_shape=jax.ShapeDtypeStruct((NSLOT, D), jnp.float32),
    )(block_expert, xs, Wg, Wu, Wd)


def kernel(x, Wr, Wg, Wu, Wd):
    b, s, d = x.shape
    xt = x.reshape(s, d)
    position, block_expert = _plan(xt, Wr)
    xs = _sc_scatter(xt, position)
    ys = _grouped_mlp(block_expert, xs, Wg, Wu, Wd)
    out = _sc_gather(ys, position, S)
    return out.reshape(b, s, d)
